# baseline (speedup 1.0000x reference)
import functools

import jax
import jax.numpy as jnp
from jax import lax
from jax.experimental import pallas as pl
from jax.experimental.pallas import tpu as pltpu
from jax.experimental.pallas import tpu_sc as plsc

_NC, _NS = 2, 16
_NW = _NC * _NS


@functools.lru_cache(maxsize=None)
def _build_gather(b, h, v, d):
    rows_per_w = b // _NW
    nbuf = 4
    assert b % _NW == 0

    mesh = plsc.VectorSubcoreMesh(core_axis_name="c", subcore_axis_name="s")

    @functools.partial(
        pl.kernel,
        mesh=mesh,
        out_type=jax.ShapeDtypeStruct((b, h, d), jnp.float32),
        scratch_types=[
            pltpu.VMEM((rows_per_w, h), jnp.int32),
            pltpu.VMEM((nbuf, h, d), jnp.float32),
            [pltpu.SemaphoreType.DMA] * nbuf,
            [pltpu.SemaphoreType.DMA] * nbuf,
        ],
        compiler_params=pltpu.CompilerParams(use_tc_tiling_on_sc=False),
    )
    def gather_kernel(idx_hbm, table_hbm, out_hbm, idx_v, rows_v, gsem, ssem):
        wid = lax.axis_index("s") * _NC + lax.axis_index("c")
        base = pl.multiple_of(wid * rows_per_w, rows_per_w)
        pltpu.sync_copy(idx_hbm.at[pl.ds(base, rows_per_w)], idx_v)

        def start_gather(r, buf):
            pltpu.async_copy(
                table_hbm.at[idx_v.at[r]], rows_v.at[buf], gsem[buf]
            )

        for buf in range(nbuf):
            start_gather(buf, buf)
        for r in range(rows_per_w):
            buf = r % nbuf
            pltpu.make_async_copy(
                table_hbm.at[idx_v.at[r]], rows_v.at[buf], gsem[buf]
            ).wait()
            pltpu.async_copy(rows_v.at[buf], out_hbm.at[base + r], ssem[buf])
            if r + nbuf < rows_per_w:
                pltpu.make_async_copy(
                    rows_v.at[buf], out_hbm.at[base + r], ssem[buf]
                ).wait()
                start_gather(r + nbuf, buf)
        for r in range(rows_per_w - nbuf, rows_per_w):
            buf = r % nbuf
            pltpu.make_async_copy(
                rows_v.at[buf], out_hbm.at[base + r], ssem[buf]
            ).wait()

    return gather_kernel


def kernel(item_ids, item_emb_weight):
    b, h = item_ids.shape
    v, d = item_emb_weight.shape
    fn = _build_gather(b, h, v, d)
    return fn(item_ids.astype(jnp.int32), item_emb_weight)
